# Initial kernel scaffold; baseline (speedup 1.0000x reference)
import jax
import jax.numpy as jnp
from jax.experimental import pallas as pl


def kernel(encoder_adj, init_emb):
    raise NotImplementedError("write your pallas kernel here")


# speedup vs baseline: 1.0083x; 1.0083x over previous
import jax
import jax.numpy as jnp
from jax.experimental import pallas as pl

N = 10000
D = 256
BM = 200


def _mm1_kernel(a_ref, x_ref, o_ref):
    o_ref[...] = jnp.dot(a_ref[...], x_ref[...],
                         preferred_element_type=jnp.float32)


def _mm2_kernel(a_ref, e1_ref, x0_ref, o2_ref, osum_ref):
    i = pl.program_id(0)
    e2 = jnp.dot(a_ref[...], e1_ref[...],
                 preferred_element_type=jnp.float32)
    o2_ref[...] = e2
    osum_ref[...] = x0_ref[...] + e1_ref[pl.ds(i * BM, BM), :] + e2


def kernel(encoder_adj, init_emb):
    grid = (N // BM,)
    a_spec = pl.BlockSpec((BM, N), lambda i: (i, 0))
    full_spec = pl.BlockSpec((N, D), lambda i: (0, 0))
    row_spec = pl.BlockSpec((BM, D), lambda i: (i, 0))

    e1 = pl.pallas_call(
        _mm1_kernel,
        grid=grid,
        in_specs=[a_spec, full_spec],
        out_specs=row_spec,
        out_shape=jax.ShapeDtypeStruct((N, D), jnp.float32),
    )(encoder_adj, init_emb)

    e2, summed = pl.pallas_call(
        _mm2_kernel,
        grid=grid,
        in_specs=[a_spec, full_spec, row_spec],
        out_specs=[row_spec, row_spec],
        out_shape=[
            jax.ShapeDtypeStruct((N, D), jnp.float32),
            jax.ShapeDtypeStruct((N, D), jnp.float32),
        ],
    )(encoder_adj, e1, init_emb)

    return (summed, init_emb, e1, e2)


# speedup vs baseline: 1.0422x; 1.0337x over previous
import jax
import jax.numpy as jnp
from jax.experimental import pallas as pl
from jax.experimental.pallas import tpu as pltpu

N = 10000
D = 256
BM = 200
NB = N // BM


def _fused_kernel(a_ref, x0_full_ref, x0_row_ref, e1_ref, e2_ref,
                  osum_ref, e1_scratch):
    i = pl.program_id(0)

    @pl.when(i < NB)
    def _():
        e1_blk = jnp.dot(a_ref[...], x0_full_ref[...],
                         preferred_element_type=jnp.float32)
        e1_ref[...] = e1_blk
        e1_scratch[pl.ds(i * BM, BM), :] = e1_blk

    @pl.when(i >= NB)
    def _():
        j = i - NB
        e2_blk = jnp.dot(a_ref[...], e1_scratch[...],
                         preferred_element_type=jnp.float32)
        e2_ref[...] = e2_blk
        osum_ref[...] = (
            x0_row_ref[...] + e1_scratch[pl.ds(j * BM, BM), :] + e2_blk)


def kernel(encoder_adj, init_emb):
    a_spec = pl.BlockSpec((BM, N), lambda i: (i % NB, 0))
    x0_full_spec = pl.BlockSpec((N, D), lambda i: (0, 0))
    x0_row_spec = pl.BlockSpec(
        (BM, D), lambda i: (jnp.maximum(i - NB, 0), 0))
    e1_spec = pl.BlockSpec((BM, D), lambda i: (jnp.minimum(i, NB - 1), 0))
    out2_spec = pl.BlockSpec((BM, D), lambda i: (jnp.maximum(i - NB, 0), 0))

    e1, e2, summed = pl.pallas_call(
        _fused_kernel,
        grid=(2 * NB,),
        in_specs=[a_spec, x0_full_spec, x0_row_spec],
        out_specs=[e1_spec, out2_spec, out2_spec],
        out_shape=[
            jax.ShapeDtypeStruct((N, D), jnp.float32),
            jax.ShapeDtypeStruct((N, D), jnp.float32),
            jax.ShapeDtypeStruct((N, D), jnp.float32),
        ],
        scratch_shapes=[pltpu.VMEM((N, D), jnp.float32)],
    )(encoder_adj, init_emb, init_emb)

    return (summed, init_emb, e1, e2)
